# Initial kernel scaffold; baseline (speedup 1.0000x reference)
import jax
import jax.numpy as jnp
from jax.experimental import pallas as pl


def kernel(input_matrix, adjacency_coo_matrix, weights_matrix, attention_bias_vector):
    raise NotImplementedError("write your pallas kernel here")


# speedup vs baseline: 16.2102x; 16.2102x over previous
import functools

import jax
import jax.numpy as jnp
from jax import lax
from jax.experimental import pallas as pl
from jax.experimental.pallas import tpu as pltpu
from jax.experimental.pallas import tpu_sc as plsc

N = 10000
E = 320000
D = 128
DP = 144
SLOPE = 0.2
NC = 2
NS = 16
NW = NC * NS
EPW = E // NW
K = 80
NCH = EPW // K
BN = 1000
GRID = N // BN
ZR = 125


def _prep_body(x_ref, w_ref, a_ref, h_ref, s_ref, m_ref):
    h = jnp.dot(x_ref[...], w_ref[...], preferred_element_type=jnp.float32)
    h_ref[:, :D] = h
    cols = lax.broadcasted_iota(jnp.int32, (BN, DP - D), 1)
    h_ref[:, D:] = jnp.where(cols == 0, 1.0, 0.0).astype(jnp.float32)
    s1 = jnp.sum(h * a_ref[0:1, :], axis=1, keepdims=True)
    s2 = jnp.sum(h * a_ref[1:2, :], axis=1, keepdims=True)
    s_ref[:, 0:1] = s1
    s_ref[:, 1:2] = s2
    mb = jnp.concatenate([jnp.full((4, 128), jnp.max(s1)),
                          jnp.full((4, 128), jnp.max(s2))], axis=0)
    i = pl.program_id(0)

    @pl.when(i == 0)
    def _():
        m_ref[...] = mb

    @pl.when(i > 0)
    def _():
        m_ref[...] = jnp.maximum(m_ref[...], mb)


_prep = pl.pallas_call(
    _prep_body,
    grid=(GRID,),
    in_specs=[
        pl.BlockSpec((BN, D), lambda i: (i, 0)),
        pl.BlockSpec((D, D), lambda i: (0, 0)),
        pl.BlockSpec((2, D), lambda i: (0, 0)),
    ],
    out_specs=[
        pl.BlockSpec((BN, DP), lambda i: (i, 0)),
        pl.BlockSpec((BN, 2), lambda i: (i, 0)),
        pl.BlockSpec((8, 128), lambda i: (0, 0)),
    ],
    out_shape=[
        jax.ShapeDtypeStruct((N, DP), jnp.float32),
        jax.ShapeDtypeStruct((N, 2), jnp.float32),
        jax.ShapeDtypeStruct((8, 128), jnp.float32),
    ],
)

_mesh = plsc.VectorSubcoreMesh(core_axis_name="c", subcore_axis_name="s",
                               num_cores=NC, num_subcores=NS)


@functools.partial(
    pl.kernel,
    out_type=jax.ShapeDtypeStruct((NC, N, DP), jnp.float32),
    mesh=_mesh,
    scratch_types=[
        pltpu.VMEM((N,), jnp.float32),
        pltpu.VMEM((N,), jnp.float32),
        pltpu.VMEM((16,), jnp.float32),
        pltpu.VMEM((K,), jnp.int32),
        pltpu.VMEM((K,), jnp.int32),
        pltpu.VMEM((K + 16,), jnp.float32),
        pltpu.VMEM((K, DP), jnp.float32),
        pltpu.VMEM_SHARED((N, DP), jnp.float32),
        pltpu.SemaphoreType.DMA,
    ],
    compiler_params=pltpu.CompilerParams(use_tc_tiling_on_sc=False,
                                         needs_layout_passes=False),
)
def _edge_kernel(h_hbm, s1_hbm, s2_hbm, src_hbm, dst_hbm, c_hbm, outp_hbm,
                 s1_v, s2_v, c_v, src_v, dst_v, a_v, rows_v,
                 out_sh, sem):
    cid = lax.axis_index("c")
    sid = lax.axis_index("s")
    wid = cid * NS + sid

    def _zb(i, carry):
        r = i // (DP // 16)
        col = (i % (DP // 16)) * 16
        rows_v[r, pl.ds(col, 16)] = jnp.zeros((16,), jnp.float32)
        return carry
    lax.fori_loop(0, K * (DP // 16), _zb, 0)
    for z in range(pl.cdiv(N // K, NS)):
        zi = sid + z * NS

        @pl.when(zi < N // K)
        def _():
            pltpu.sync_copy(rows_v, out_sh.at[pl.ds(zi * K, K), :])

    pltpu.sync_copy(s1_hbm, s1_v)
    pltpu.sync_copy(s2_hbm, s2_v)
    pltpu.sync_copy(c_hbm, c_v)
    plsc.subcore_barrier()

    ebase = wid * EPW

    def _chunk(j, carry):
        e0 = ebase + j * K
        pltpu.sync_copy(src_hbm.at[pl.ds(e0, K)], src_v)
        pltpu.sync_copy(dst_hbm.at[pl.ds(e0, K)], dst_v)
        pltpu.async_copy(h_hbm.at[dst_v], rows_v, sem).wait()
        cvec = c_v[...]
        for u in range(K // 16):
            sv = src_v[pl.ds(u * 16, 16)]
            dv = dst_v[pl.ds(u * 16, 16)]
            al = plsc.load_gather(s1_v, [sv]) + plsc.load_gather(s2_v, [dv])
            al = jnp.where(al > 0, al, al * SLOPE)
            a_v[pl.ds(u * 16, 16)] = jnp.exp(al - cvec)

        def _scale(r, c2):
            av = a_v[pl.ds(r, 16)][0]
            for q in range(DP // 16):
                rows_v[r, pl.ds(q * 16, 16)] = rows_v[r, pl.ds(q * 16, 16)] * av
            return c2
        lax.fori_loop(0, K, _scale, 0)
        pltpu.sync_copy(rows_v, out_sh.at[src_v], add=True)
        return carry

    lax.fori_loop(0, NCH, _chunk, 0)

    plsc.subcore_barrier()

    @pl.when(sid == 0)
    def _():
        pltpu.sync_copy(out_sh, outp_hbm.at[cid])


def _final_body(p_ref, h_ref, s_ref, c_ref, o_ref):
    al = s_ref[:, 0:1] + s_ref[:, 1:2]
    al = jnp.where(al > 0, al, al * SLOPE)
    asf = jnp.exp(al - c_ref[0, 0])
    h = h_ref[:, :D]
    num = p_ref[0, :, :D] + p_ref[1, :, :D] + asf * h
    den = p_ref[0, :, D:D + 1] + p_ref[1, :, D:D + 1] + asf
    o_ref[...] = num / den


_final = pl.pallas_call(
    _final_body,
    grid=(GRID,),
    in_specs=[
        pl.BlockSpec((NC, BN, DP), lambda i: (0, i, 0)),
        pl.BlockSpec((BN, DP), lambda i: (i, 0)),
        pl.BlockSpec((BN, 2), lambda i: (i, 0)),
        pl.BlockSpec(memory_space=pltpu.SMEM),
    ],
    out_specs=pl.BlockSpec((BN, D), lambda i: (i, 0)),
    out_shape=jax.ShapeDtypeStruct((N, D), jnp.float32),
)


def kernel(input_matrix, adjacency_coo_matrix, weights_matrix,
           attention_bias_vector):
    adj = adjacency_coo_matrix.astype(jnp.int32)
    src = adj[0]
    dst = adj[1]
    a2 = attention_bias_vector.reshape(2, D)
    h_pad, s, m = _prep(input_matrix, weights_matrix, a2)
    c = m[0, 0] + m[4, 0]
    c16 = jnp.full((16,), c, jnp.float32)
    c11 = c.reshape(1, 1)
    outp = _edge_kernel(h_pad, s[:, 0], s[:, 1], src, dst, c16)
    return _final(outp, h_pad, s, c11)


# speedup vs baseline: 28.5458x; 1.7610x over previous
import functools

import jax
import jax.numpy as jnp
from jax import lax
from jax.experimental import pallas as pl
from jax.experimental.pallas import tpu as pltpu
from jax.experimental.pallas import tpu_sc as plsc

N = 10000
E = 320000
D = 128
DP = 144
SLOPE = 0.2
NC = 2
NS = 16
NW = NC * NS
EPW = E // NW
K = 80
NCH = EPW // K
IB = 2000
CPB = IB // K
NB = EPW // IB
BN = 1000
GRID = N // BN


def _prep_body(x_ref, w_ref, a_ref, h_ref, s_ref, m_ref):
    h = jnp.dot(x_ref[...], w_ref[...], preferred_element_type=jnp.float32)
    h_ref[:, :D] = h
    cols = lax.broadcasted_iota(jnp.int32, (BN, DP - D), 1)
    h_ref[:, D:] = jnp.where(cols == 0, 1.0, 0.0).astype(jnp.float32)
    s1 = jnp.sum(h * a_ref[0:1, :], axis=1, keepdims=True)
    s2 = jnp.sum(h * a_ref[1:2, :], axis=1, keepdims=True)
    s_ref[:, 0:1] = s1
    s_ref[:, 1:2] = s2
    mb = jnp.concatenate([jnp.full((4, 128), jnp.max(s1)),
                          jnp.full((4, 128), jnp.max(s2))], axis=0)
    i = pl.program_id(0)

    @pl.when(i == 0)
    def _():
        m_ref[...] = mb

    @pl.when(i > 0)
    def _():
        m_ref[...] = jnp.maximum(m_ref[...], mb)


_prep = pl.pallas_call(
    _prep_body,
    grid=(GRID,),
    in_specs=[
        pl.BlockSpec((BN, D), lambda i: (i, 0)),
        pl.BlockSpec((D, D), lambda i: (0, 0)),
        pl.BlockSpec((2, D), lambda i: (0, 0)),
    ],
    out_specs=[
        pl.BlockSpec((BN, DP), lambda i: (i, 0)),
        pl.BlockSpec((BN, 2), lambda i: (i, 0)),
        pl.BlockSpec((8, 128), lambda i: (0, 0)),
    ],
    out_shape=[
        jax.ShapeDtypeStruct((N, DP), jnp.float32),
        jax.ShapeDtypeStruct((N, 2), jnp.float32),
        jax.ShapeDtypeStruct((8, 128), jnp.float32),
    ],
)

_mesh = plsc.VectorSubcoreMesh(core_axis_name="c", subcore_axis_name="s",
                               num_cores=NC, num_subcores=NS)
_sc_params = pltpu.CompilerParams(use_tc_tiling_on_sc=False,
                                  needs_layout_passes=False)


@functools.partial(
    pl.kernel,
    out_type=jax.ShapeDtypeStruct((E,), jnp.float32),
    mesh=_mesh,
    scratch_types=[
        pltpu.VMEM((N,), jnp.float32),
        pltpu.VMEM((N,), jnp.float32),
        pltpu.VMEM((16,), jnp.float32),
        pltpu.VMEM((IB,), jnp.int32),
        pltpu.VMEM((IB,), jnp.int32),
        pltpu.VMEM((IB,), jnp.float32),
    ],
    compiler_params=_sc_params,
)
def _coef_kernel(s1_hbm, s2_hbm, src_hbm, dst_hbm, c_hbm, ea_hbm,
                 s1_v, s2_v, c_v, src_v, dst_v, a_v):
    cid = lax.axis_index("c")
    sid = lax.axis_index("s")
    wid = cid * NS + sid
    tbase = wid * EPW

    pltpu.sync_copy(s1_hbm, s1_v)
    pltpu.sync_copy(s2_hbm, s2_v)
    pltpu.sync_copy(c_hbm, c_v)
    cvec = c_v[...]

    def _block(b, carry):
        e0 = tbase + b * IB
        pltpu.sync_copy(src_hbm.at[pl.ds(e0, IB)], src_v)
        pltpu.sync_copy(dst_hbm.at[pl.ds(e0, IB)], dst_v)

        def _grp(u, c2):
            sv = src_v[pl.ds(u * 16, 16)]
            dv = dst_v[pl.ds(u * 16, 16)]
            al = plsc.load_gather(s1_v, [sv]) + plsc.load_gather(s2_v, [dv])
            al = jnp.where(al > 0, al, al * SLOPE)
            a_v[pl.ds(u * 16, 16)] = jnp.exp(al - cvec)
            return c2
        lax.fori_loop(0, IB // 16, _grp, 0)
        pltpu.sync_copy(a_v, ea_hbm.at[pl.ds(e0, IB)])
        return carry

    lax.fori_loop(0, NB, _block, 0)


@functools.partial(
    pl.kernel,
    out_type=jax.ShapeDtypeStruct((NC, N, DP), jnp.float32),
    mesh=_mesh,
    scratch_types=[
        pltpu.VMEM((NCH, K), jnp.int32),
        pltpu.VMEM((CPB, K), jnp.int32),
        pltpu.VMEM((IB + 16,), jnp.float32),
        pltpu.VMEM((K, DP), jnp.float32),
        pltpu.VMEM((K, DP), jnp.float32),
        pltpu.VMEM_SHARED((N, DP), jnp.float32),
        pltpu.SemaphoreType.DMA,
        pltpu.SemaphoreType.DMA,
    ],
    compiler_params=_sc_params,
)
def _edge_kernel(h_hbm, src2_hbm, dst2_hbm, ea_hbm, outp_hbm,
                 dst2_v, src2_v, a_v, rows0_v, rows1_v, out_sh,
                 gsem0, gsem1):
    cid = lax.axis_index("c")
    sid = lax.axis_index("s")
    wid = cid * NS + sid

    def _zb(i, carry):
        r = i // (DP // 16)
        col = (i % (DP // 16)) * 16
        rows0_v[r, pl.ds(col, 16)] = jnp.zeros((16,), jnp.float32)
        return carry
    lax.fori_loop(0, K * (DP // 16), _zb, 0)
    for z in range(pl.cdiv(N // K, NS)):
        zi = sid + z * NS

        @pl.when(zi < N // K)
        def _():
            pltpu.sync_copy(rows0_v, out_sh.at[pl.ds(zi * K, K), :])

    pltpu.sync_copy(dst2_hbm.at[pl.ds(wid * NCH, NCH), :], dst2_v)
    plsc.subcore_barrier()

    pltpu.async_copy(h_hbm.at[dst2_v.at[0]], rows0_v, gsem0)

    def _proc(j, jj, rows_cur, gsem_cur, rows_nxt, gsem_nxt):
        @pl.when(j + 1 < NCH)
        def _():
            pltpu.async_copy(h_hbm.at[dst2_v.at[j + 1]], rows_nxt, gsem_nxt)
        pltpu.make_async_copy(h_hbm.at[dst2_v.at[j]], rows_cur,
                              gsem_cur).wait()

        def _scale(r, c2):
            av = a_v[pl.ds(jj * K + r, 16)][0]
            for q in range(DP // 16):
                rows_cur[r, pl.ds(q * 16, 16)] = \
                    rows_cur[r, pl.ds(q * 16, 16)] * av
            return c2
        lax.fori_loop(0, K, _scale, 0)
        pltpu.sync_copy(rows_cur, out_sh.at[src2_v.at[jj]], add=True)

    def _block(b, carry):
        pltpu.sync_copy(src2_hbm.at[pl.ds(wid * NCH + b * CPB, CPB), :],
                        src2_v)
        pltpu.sync_copy(ea_hbm.at[pl.ds(wid * EPW + b * IB, IB)],
                        a_v.at[pl.ds(0, IB)])

        def _chunk(jj, c2):
            j = b * CPB + jj
            parity = lax.rem(j, 2)

            @pl.when(parity == 0)
            def _():
                _proc(j, jj, rows0_v, gsem0, rows1_v, gsem1)

            @pl.when(parity == 1)
            def _():
                _proc(j, jj, rows1_v, gsem1, rows0_v, gsem0)
            return c2
        lax.fori_loop(0, CPB, _chunk, 0)
        return carry

    lax.fori_loop(0, NB, _block, 0)

    plsc.subcore_barrier()

    @pl.when(sid == 0)
    def _():
        pltpu.sync_copy(out_sh, outp_hbm.at[cid])


def _final_body(p_ref, h_ref, s_ref, c_ref, o_ref):
    al = s_ref[:, 0:1] + s_ref[:, 1:2]
    al = jnp.where(al > 0, al, al * SLOPE)
    asf = jnp.exp(al - c_ref[0, 0])
    h = h_ref[:, :D]
    num = p_ref[0, :, :D] + p_ref[1, :, :D] + asf * h
    den = p_ref[0, :, D:D + 1] + p_ref[1, :, D:D + 1] + asf
    o_ref[...] = num / den


_final = pl.pallas_call(
    _final_body,
    grid=(GRID,),
    in_specs=[
        pl.BlockSpec((NC, BN, DP), lambda i: (0, i, 0)),
        pl.BlockSpec((BN, DP), lambda i: (i, 0)),
        pl.BlockSpec((BN, 2), lambda i: (i, 0)),
        pl.BlockSpec(memory_space=pltpu.SMEM),
    ],
    out_specs=pl.BlockSpec((BN, D), lambda i: (i, 0)),
    out_shape=jax.ShapeDtypeStruct((N, D), jnp.float32),
)


def kernel(input_matrix, adjacency_coo_matrix, weights_matrix,
           attention_bias_vector):
    adj = adjacency_coo_matrix.astype(jnp.int32)
    src = adj[0]
    dst = adj[1]
    a2 = attention_bias_vector.reshape(2, D)
    h_pad, s, m = _prep(input_matrix, weights_matrix, a2)
    c = m[0, 0] + m[4, 0]
    c16 = jnp.full((16,), c, jnp.float32)
    c11 = c.reshape(1, 1)
    ea = _coef_kernel(s[:, 0], s[:, 1], src, dst, c16)
    src2 = src.reshape(NW * NCH, K)
    dst2 = dst.reshape(NW * NCH, K)
    outp = _edge_kernel(h_pad, src2, dst2, ea)
    return _final(outp, h_pad, s, c11)
